# baseline (speedup 1.0000x reference)
import functools

import jax
import jax.numpy as jnp
from jax import lax
from jax.experimental import pallas as pl
from jax.experimental.pallas import tpu as pltpu
from jax.experimental.pallas import tpu_sc as plsc

D = 32
NBLK = 1954
WR = NBLK * 128


def _repack(Zt):
    def tk(zt_ref, w_ref):
        for jj in range(4):
            w_ref[:, 32 * jj:32 * jj + 32] = (
                zt_ref[:, 128 * jj:128 * jj + 128].T
            )

    return pl.pallas_call(
        tk,
        grid=(NBLK,),
        in_specs=[pl.BlockSpec((D, 512), lambda b: (0, b))],
        out_specs=pl.BlockSpec((128, 128), lambda b: (b, 0)),
        out_shape=jax.ShapeDtypeStruct((WR, 128), jnp.float32),
    )(Zt)


def kernel(indices, Z):
    B = indices.shape[0]
    info = plsc.get_sparse_core_info()
    NC, NS = info.num_cores, info.num_subcores
    NW = NC * NS
    BW = B // NW

    mesh = plsc.VectorSubcoreMesh(core_axis_name="c", subcore_axis_name="s")
    Zt = Z.T
    W = _repack(Zt)

    @functools.partial(
        pl.kernel,
        mesh=mesh,
        compiler_params=pltpu.CompilerParams(needs_layout_passes=False),
        out_type=jax.ShapeDtypeStruct((D, B), jnp.float32),
        scratch_types=[
            pltpu.VMEM((BW,), jnp.int32),
            pltpu.VMEM((4, 128), jnp.int32),
            pltpu.VMEM((BW, 128), jnp.float32),
            pltpu.VMEM((D, BW), jnp.float32),
            pltpu.SemaphoreType.DMA,
        ],
    )
    def gather_kernel(idx_hbm, w_hbm, out_hbm, idx_v, kidx, rows, stg, sem):
        wid = lax.axis_index("s") * NC + lax.axis_index("c")
        base = pl.multiple_of(wid * BW, 128)
        pltpu.sync_copy(idx_hbm.at[pl.ds(base, BW)], idx_v)
        row0 = lax.broadcasted_iota(jnp.int32, (16,), 0)
        row1 = row0 + 16

        for t in range(0, BW, 16):
            iv = idx_v[pl.ds(t, 16)]
            kidx[t // 128, pl.ds(t % 128, 16)] = ((iv >> 9) << 7) | (iv & 127)

        copies = [
            pltpu.async_copy(
                w_hbm.at[kidx.at[j]], rows.at[pl.ds(128 * j, 128)], sem
            )
            for j in range(4)
        ]
        for c in copies:
            c.wait()

        @pl.loop(0, BW, step=16)
        def _extract(t):
            iv = idx_v[pl.ds(t, 16)]
            for s in range(16):
                i0 = iv[s]
                q = pl.multiple_of(((i0 >> 7) & 3) << 5, 32)
                r = rows.at[t + s]
                v0 = r[pl.ds(q, 16)]
                v1 = r[pl.ds(q + 16, 16)]
                pvec = jnp.broadcast_to(t + s, (16,))
                plsc.store_scatter(stg, [row0, pvec], v0)
                plsc.store_scatter(stg, [row1, pvec], v1)

        pltpu.sync_copy(stg, out_hbm.at[:, pl.ds(base, BW)])

    return gather_kernel(indices, W).T
